# Initial kernel scaffold; baseline (speedup 1.0000x reference)
import jax
import jax.numpy as jnp
from jax.experimental import pallas as pl


def kernel(x, gate_w, w_gate, w_up, w_down):
    raise NotImplementedError("write your pallas kernel here")


# speedup vs baseline: 3.1587x; 3.1587x over previous
import functools

import jax
import jax.numpy as jnp
from jax import lax
from jax.experimental import pallas as pl
from jax.experimental.pallas import tpu as pltpu
from jax.experimental.pallas import tpu_sc as plsc

E = 8
K = 2
D = 1024
F = 4096
TM = 128
RB = 256
LANES = 128

NC = 2
NS = 16
NW = NC * NS

_mesh = plsc.VectorSubcoreMesh(core_axis_name="c", subcore_axis_name="s")


def _router_body(x_ref, gw_ref, s_ref, e_ref):
    x = x_ref[...]
    gw = gw_ref[...]
    logits = jnp.dot(x, gw, preferred_element_type=jnp.float32)
    lane = lax.broadcasted_iota(jnp.int32, logits.shape, 1)
    valid = lane < E
    neg = jnp.float32(-1e30)
    l = jnp.where(valid, logits, neg)
    m = jnp.max(l, axis=1, keepdims=True)
    p = jnp.where(valid, jnp.exp(l - m), 0.0)
    s = p / jnp.sum(p, axis=1, keepdims=True)
    big = jnp.int32(LANES * 2)
    m1 = jnp.max(s, axis=1, keepdims=True)
    e1 = jnp.min(jnp.where((s == m1) & valid, lane, big), axis=1, keepdims=True)
    s_wo1 = jnp.where(lane == e1, -1.0, jnp.where(valid, s, -1.0))
    m2 = jnp.max(s_wo1, axis=1, keepdims=True)
    e2 = jnp.min(jnp.where((s_wo1 == m2) & valid, lane, big), axis=1, keepdims=True)
    s_ref[...] = jnp.where(lane == 0, m1, jnp.where(lane == 1, m2, 0.0))
    e_ref[...] = jnp.where(lane == 0, e1, jnp.where(lane == 1, e2, 0))


def _router(x_flat, gw_pad, n):
    return pl.pallas_call(
        _router_body,
        grid=(n // RB,),
        in_specs=[
            pl.BlockSpec((RB, D), lambda i: (i, 0)),
            pl.BlockSpec((D, LANES), lambda i: (0, 0)),
        ],
        out_specs=[
            pl.BlockSpec((RB, LANES), lambda i: (i, 0)),
            pl.BlockSpec((RB, LANES), lambda i: (i, 0)),
        ],
        out_shape=[
            jax.ShapeDtypeStruct((n, LANES), jnp.float32),
            jax.ShapeDtypeStruct((n, LANES), jnp.int32),
        ],
    )(x_flat, gw_pad)


def _make_sc_gather(p_rows, ch):
    rows_w = p_rows // NW

    @functools.partial(
        pl.kernel,
        out_type=jax.ShapeDtypeStruct((p_rows, D), jnp.float32),
        mesh=_mesh,
        scratch_types=[
            pltpu.VMEM((rows_w,), jnp.int32),
            pltpu.VMEM((ch, D), jnp.float32),
            pltpu.SemaphoreType.DMA,
        ],
    )
    def gather(x_hbm, idx_hbm, out_hbm, idx_v, rows_v, sem):
        wid = lax.axis_index("s") * NC + lax.axis_index("c")
        base = wid * rows_w
        pltpu.sync_copy(idx_hbm.at[pl.ds(base, rows_w)], idx_v)
        for c in range(rows_w // ch):
            pltpu.async_copy(
                x_hbm.at[idx_v.at[pl.ds(c * ch, ch)]], rows_v, sem
            ).wait()
            pltpu.sync_copy(rows_v, out_hbm.at[pl.ds(base + c * ch, ch)])

    return gather


def _ffn_body(te_ref, xg_ref, s_ref, wg_ref, wu_ref, wd_ref, out_ref):
    del te_ref
    xs = (xg_ref[...] * s_ref[:, 0:1]).astype(jnp.bfloat16)
    wg = jnp.reshape(wg_ref[...], (D, F))
    wu = jnp.reshape(wu_ref[...], (D, F))
    wd = jnp.reshape(wd_ref[...], (F, D))
    g = jnp.dot(xs, wg, preferred_element_type=jnp.float32)
    u = jnp.dot(xs, wu, preferred_element_type=jnp.float32)
    h = (g * (1.0 / (1.0 + jnp.exp(-g))) * u).astype(jnp.bfloat16)
    out_ref[...] = jnp.dot(h, wd, preferred_element_type=jnp.float32)


def _ffn(te, xg, s_b, w_gate, w_up, w_down, p_rows):
    t = p_rows // TM
    return pl.pallas_call(
        _ffn_body,
        grid_spec=pltpu.PrefetchScalarGridSpec(
            num_scalar_prefetch=1,
            grid=(t,),
            in_specs=[
                pl.BlockSpec((TM, D), lambda i, te_r: (i, 0)),
                pl.BlockSpec((TM, LANES), lambda i, te_r: (i, 0)),
                pl.BlockSpec((1, D, F), lambda i, te_r: (te_r[i], 0, 0)),
                pl.BlockSpec((1, D, F), lambda i, te_r: (te_r[i], 0, 0)),
                pl.BlockSpec((1, F, D), lambda i, te_r: (te_r[i], 0, 0)),
            ],
            out_specs=pl.BlockSpec((TM, D), lambda i, te_r: (i, 0)),
        ),
        out_shape=jax.ShapeDtypeStruct((p_rows, D), jnp.float32),
    )(te, xg, s_b, w_gate, w_up, w_down)


def _make_sc_combine(n, cc):
    tok_w = n // NW

    @functools.partial(
        pl.kernel,
        out_type=jax.ShapeDtypeStruct((n, D), jnp.float32),
        mesh=_mesh,
        scratch_types=[
            pltpu.VMEM((tok_w,), jnp.int32),
            pltpu.VMEM((tok_w,), jnp.int32),
            pltpu.VMEM((cc, D), jnp.float32),
            pltpu.VMEM((cc, D), jnp.float32),
            pltpu.SemaphoreType.DMA,
            pltpu.SemaphoreType.DMA,
        ],
    )
    def combine(y_hbm, c0_hbm, c1_hbm, out_hbm, i0_v, i1_v, a_v, b_v, s0, s1):
        wid = lax.axis_index("s") * NC + lax.axis_index("c")
        base = wid * tok_w
        pltpu.sync_copy(c0_hbm.at[pl.ds(base, tok_w)], i0_v)
        pltpu.sync_copy(c1_hbm.at[pl.ds(base, tok_w)], i1_v)
        for c in range(tok_w // cc):
            cp0 = pltpu.async_copy(y_hbm.at[i0_v.at[pl.ds(c * cc, cc)]], a_v, s0)
            cp1 = pltpu.async_copy(y_hbm.at[i1_v.at[pl.ds(c * cc, cc)]], b_v, s1)
            cp0.wait()
            cp1.wait()

            def add_row(r, carry):
                for k in range(D // 16):
                    sl = pl.ds(k * 16, 16)
                    a_v[r, sl] = a_v[r, sl] + b_v[r, sl]
                return carry

            lax.fori_loop(0, cc, add_row, 0)
            pltpu.sync_copy(a_v, out_hbm.at[pl.ds(base + c * cc, cc)])

    return combine


def kernel(x, gate_w, w_gate, w_up, w_down):
    bs, slen, d = x.shape
    n = bs * slen
    s_slots = n * K
    p_rows = s_slots + E * TM
    x_flat = x.reshape(n, d)

    gw_pad = jnp.zeros((d, LANES), jnp.float32).at[:, :E].set(gate_w)
    srt, idt = _router(x_flat, gw_pad, n)
    sco = srt[:, :K].reshape(-1)
    sel = idt[:, :K].reshape(-1)

    oh = (sel[:, None] == jnp.arange(E, dtype=jnp.int32)[None, :]).astype(jnp.int32)
    within = jnp.take_along_axis(jnp.cumsum(oh, axis=0), sel[:, None], axis=1)[:, 0] - 1
    counts = jnp.sum(oh, axis=0)
    pc = ((counts + TM - 1) // TM) * TM
    ends = jnp.cumsum(pc)
    starts = ends - pc
    rank = (starts[sel] + within).astype(jnp.int32)
    tok = (jnp.arange(s_slots, dtype=jnp.int32) // K)
    tok_sorted = jnp.zeros((p_rows,), jnp.int32).at[rank].set(tok)
    s_sorted = jnp.zeros((p_rows,), jnp.float32).at[rank].set(sco)
    tile_start = jnp.arange(p_rows // TM, dtype=jnp.int32) * TM
    te = jnp.sum((tile_start[:, None] >= ends[None, :]).astype(jnp.int32), axis=1)
    te = jnp.clip(te, 0, E - 1).astype(jnp.int32)

    xg = _make_sc_gather(p_rows, 96)(x_flat, tok_sorted)

    s_b = jnp.broadcast_to(s_sorted[:, None], (p_rows, LANES))
    y = _ffn(te, xg, s_b,
             w_gate.astype(jnp.bfloat16),
             w_up.astype(jnp.bfloat16),
             w_down.astype(jnp.bfloat16), p_rows)

    cidx = rank.reshape(n, K)
    out = _make_sc_combine(n, 32)(y, cidx[:, 0], cidx[:, 1])
    return out.reshape(bs, slen, d)


# speedup vs baseline: 4.0790x; 1.2913x over previous
import functools

import jax
import jax.numpy as jnp
from jax import lax
from jax.experimental import pallas as pl
from jax.experimental.pallas import tpu as pltpu
from jax.experimental.pallas import tpu_sc as plsc

E = 8
K = 2
D = 1024
F = 4096
FB = F // 2
TM = 128
RB = 512
LANES = 128

NC = 2
NS = 16
NW = NC * NS
CH = 32

_mesh = plsc.VectorSubcoreMesh(core_axis_name="c", subcore_axis_name="s")


def _router_body(x_ref, gw_ref, s_ref, e_ref):
    x = x_ref[...]
    gw = gw_ref[...]
    logits = jnp.dot(x, gw, preferred_element_type=jnp.float32)
    lane = lax.broadcasted_iota(jnp.int32, logits.shape, 1)
    valid = lane < E
    neg = jnp.float32(-1e30)
    l = jnp.where(valid, logits, neg)
    m = jnp.max(l, axis=1, keepdims=True)
    p = jnp.where(valid, jnp.exp(l - m), 0.0)
    s = p / jnp.sum(p, axis=1, keepdims=True)
    big = jnp.int32(LANES * 2)
    m1 = jnp.max(s, axis=1, keepdims=True)
    e1 = jnp.min(jnp.where((s == m1) & valid, lane, big), axis=1, keepdims=True)
    s_wo1 = jnp.where(lane == e1, -1.0, jnp.where(valid, s, -1.0))
    m2 = jnp.max(s_wo1, axis=1, keepdims=True)
    e2 = jnp.min(jnp.where((s_wo1 == m2) & valid, lane, big), axis=1, keepdims=True)
    s_ref[...] = jnp.where(lane == 0, m1, jnp.where(lane == 1, m2, 0.0))
    e_ref[...] = jnp.where(lane == 0, e1, jnp.where(lane == 1, e2, 0))


def _router(x_flat, gw_pad, n):
    return pl.pallas_call(
        _router_body,
        grid=(n // RB,),
        in_specs=[
            pl.BlockSpec((RB, D), lambda i: (i, 0)),
            pl.BlockSpec((D, LANES), lambda i: (0, 0)),
        ],
        out_specs=[
            pl.BlockSpec((RB, LANES), lambda i: (i, 0)),
            pl.BlockSpec((RB, LANES), lambda i: (i, 0)),
        ],
        out_shape=[
            jax.ShapeDtypeStruct((n, LANES), jnp.float32),
            jax.ShapeDtypeStruct((n, LANES), jnp.int32),
        ],
    )(x_flat, gw_pad)


def _make_sc_permute(s_slots, p_rows):
    slots_w = s_slots // NW
    nch = slots_w // CH

    @functools.partial(
        pl.kernel,
        out_type=[
            jax.ShapeDtypeStruct((p_rows, D), jnp.float32),
            jax.ShapeDtypeStruct((p_rows, LANES), jnp.float32),
        ],
        mesh=_mesh,
        scratch_types=[
            pltpu.VMEM((nch, CH), jnp.int32),
            pltpu.VMEM((slots_w,), jnp.int32),
            pltpu.VMEM((CH, D), jnp.float32),
            pltpu.VMEM((CH, D), jnp.float32),
            pltpu.VMEM((CH, LANES), jnp.float32),
            pltpu.VMEM((CH, LANES), jnp.float32),
        ]
        + [pltpu.SemaphoreType.DMA] * 8,
    )
    def permute(x_hbm, rank_hbm, tok_hbm, sb_hbm, xg_hbm, sp_hbm,
                idx_v, tok_v, r0, r1, q0, q1,
                g0, g1, h0, h1, ox0, ox1, os0, os1):
        wid = lax.axis_index("s") * NC + lax.axis_index("c")
        j0 = wid * slots_w
        pltpu.sync_copy(rank_hbm.at[wid], idx_v)
        pltpu.sync_copy(tok_hbm.at[pl.ds(j0, slots_w)], tok_v)
        rbuf = (r0, r1)
        qbuf = (q0, q1)
        gsem = (g0, g1)
        hsem = (h0, h1)
        oxsem = (ox0, ox1)
        ossem = (os0, os1)

        def start_in(c, b):
            gh = pltpu.async_copy(
                x_hbm.at[tok_v.at[pl.ds(c * CH, CH)]], rbuf[b], gsem[b])
            sh = pltpu.async_copy(
                sb_hbm.at[pl.ds(j0 + c * CH, CH)], qbuf[b], hsem[b])
            return gh, sh

        pend = [start_in(0, 0), start_in(1, 1)]
        out_pend = [None, None]
        for c in range(nch):
            b = c % 2
            gh, sh = pend[b]
            gh.wait()
            sh.wait()
            oh = (
                pltpu.async_copy(rbuf[b], xg_hbm.at[idx_v.at[c]], oxsem[b]),
                pltpu.async_copy(qbuf[b], sp_hbm.at[idx_v.at[c]], ossem[b]),
            )
            out_pend[b] = oh
            if c + 2 < nch:
                oh[0].wait()
                oh[1].wait()
                out_pend[b] = None
                pend[b] = start_in(c + 2, b)
        for b in range(2):
            if out_pend[b] is not None:
                out_pend[b][0].wait()
                out_pend[b][1].wait()

    return permute


def _ffn_a_body(te_ref, xg_ref, s_ref, wg_ref, wu_ref, h_ref):
    del te_ref
    xs = (xg_ref[...] * s_ref[:, 0:1]).astype(jnp.bfloat16)
    wg = jnp.reshape(wg_ref[...], (D, FB)).astype(jnp.bfloat16)
    wu = jnp.reshape(wu_ref[...], (D, FB)).astype(jnp.bfloat16)
    g = jnp.dot(xs, wg, preferred_element_type=jnp.float32)
    u = jnp.dot(xs, wu, preferred_element_type=jnp.float32)
    h_ref[...] = (g * (1.0 / (1.0 + jnp.exp(-g))) * u).astype(jnp.bfloat16)


def _ffn_b_body(te_ref, h_ref, wd_ref, out_ref):
    del te_ref
    h = h_ref[...]
    wd = jnp.reshape(wd_ref[...], (F, D)).astype(jnp.bfloat16)
    out_ref[...] = jnp.dot(h, wd, preferred_element_type=jnp.float32)


def _ffn(te, xg, s_p, w_gate, w_up, w_down, p_rows):
    t = p_rows // TM
    h = pl.pallas_call(
        _ffn_a_body,
        grid_spec=pltpu.PrefetchScalarGridSpec(
            num_scalar_prefetch=1,
            grid=(F // FB, t),
            in_specs=[
                pl.BlockSpec((TM, D), lambda j, i, te_r: (i, 0)),
                pl.BlockSpec((TM, LANES), lambda j, i, te_r: (i, 0)),
                pl.BlockSpec((1, D, FB), lambda j, i, te_r: (te_r[i], 0, j)),
                pl.BlockSpec((1, D, FB), lambda j, i, te_r: (te_r[i], 0, j)),
            ],
            out_specs=pl.BlockSpec((TM, FB), lambda j, i, te_r: (i, j)),
        ),
        out_shape=jax.ShapeDtypeStruct((p_rows, F), jnp.bfloat16),
    )(te, xg, s_p, w_gate, w_up)
    return pl.pallas_call(
        _ffn_b_body,
        grid_spec=pltpu.PrefetchScalarGridSpec(
            num_scalar_prefetch=1,
            grid=(t,),
            in_specs=[
                pl.BlockSpec((TM, F), lambda i, te_r: (i, 0)),
                pl.BlockSpec((1, F, D), lambda i, te_r: (te_r[i], 0, 0)),
            ],
            out_specs=pl.BlockSpec((TM, D), lambda i, te_r: (i, 0)),
        ),
        out_shape=jax.ShapeDtypeStruct((p_rows, D), jnp.float32),
    )(te, h, w_down)


def _make_sc_combine(n, cc):
    tok_w = n // NW

    @functools.partial(
        pl.kernel,
        out_type=jax.ShapeDtypeStruct((n, D), jnp.float32),
        mesh=_mesh,
        scratch_types=[
            pltpu.VMEM((tok_w,), jnp.int32),
            pltpu.VMEM((tok_w,), jnp.int32),
            pltpu.VMEM((cc, D), jnp.float32),
            pltpu.VMEM((cc, D), jnp.float32),
            pltpu.SemaphoreType.DMA,
            pltpu.SemaphoreType.DMA,
        ],
    )
    def combine(y_hbm, c0_hbm, c1_hbm, out_hbm, i0_v, i1_v, a_v, b_v, s0, s1):
        wid = lax.axis_index("s") * NC + lax.axis_index("c")
        base = wid * tok_w
        pltpu.sync_copy(c0_hbm.at[pl.ds(base, tok_w)], i0_v)
        pltpu.sync_copy(c1_hbm.at[pl.ds(base, tok_w)], i1_v)
        for c in range(tok_w // cc):
            cp0 = pltpu.async_copy(y_hbm.at[i0_v.at[pl.ds(c * cc, cc)]], a_v, s0)
            cp1 = pltpu.async_copy(y_hbm.at[i1_v.at[pl.ds(c * cc, cc)]], b_v, s1)
            cp0.wait()
            cp1.wait()

            def add_row(r, carry):
                for k in range(D // 16):
                    sl = pl.ds(k * 16, 16)
                    a_v[r, sl] = a_v[r, sl] + b_v[r, sl]
                return carry

            lax.fori_loop(0, cc, add_row, 0)
            pltpu.sync_copy(a_v, out_hbm.at[pl.ds(base + c * cc, cc)])

    return combine


def kernel(x, gate_w, w_gate, w_up, w_down):
    bs, slen, d = x.shape
    n = bs * slen
    s_slots = n * K
    p_rows = s_slots + E * TM
    x_flat = x.reshape(n, d)

    gw_pad = jnp.zeros((d, LANES), jnp.float32).at[:, :E].set(gate_w)
    srt, idt = _router(x_flat, gw_pad, n)
    sco = srt[:, :K].reshape(-1)
    sel = idt[:, :K].reshape(-1)

    oh = (sel[:, None] == jnp.arange(E, dtype=jnp.int32)[None, :]).astype(jnp.int32)
    within = jnp.take_along_axis(jnp.cumsum(oh, axis=0), sel[:, None], axis=1)[:, 0] - 1
    counts = jnp.sum(oh, axis=0)
    pc = ((counts + TM - 1) // TM) * TM
    ends = jnp.cumsum(pc)
    starts = ends - pc
    rank = (starts[sel] + within).astype(jnp.int32)
    tok = (jnp.arange(s_slots, dtype=jnp.int32) // K)
    tile_start = jnp.arange(p_rows // TM, dtype=jnp.int32) * TM
    te = jnp.sum((tile_start[:, None] >= ends[None, :]).astype(jnp.int32), axis=1)
    te = jnp.clip(te, 0, E - 1).astype(jnp.int32)

    rank3 = rank.reshape(NW, (s_slots // NW) // CH, CH)
    sb = jnp.broadcast_to(sco[:, None], (s_slots, LANES))
    xg, s_p = _make_sc_permute(s_slots, p_rows)(x_flat, rank3, tok, sb)

    y = _ffn(te, xg, s_p, w_gate, w_up, w_down, p_rows)

    cidx = rank.reshape(n, K)
    out = _make_sc_combine(n, 32)(y, cidx[:, 0], cidx[:, 1])
    return out.reshape(bs, slen, d)
